# Initial kernel scaffold; baseline (speedup 1.0000x reference)
import jax
import jax.numpy as jnp
from jax.experimental import pallas as pl


def kernel(inputs, word_table, pos_table):
    raise NotImplementedError("write your pallas kernel here")


# speedup vs baseline: 1.3931x; 1.3931x over previous
import functools

import jax
import jax.numpy as jnp
from jax import lax
from jax.experimental import pallas as pl
from jax.experimental.pallas import tpu as pltpu
from jax.experimental.pallas import tpu_sc as plsc

_BATCH = 4096
_SEQ = 200
_DIM = 32
_NW = 32
_TOTAL = _BATCH * _SEQ
_PER_W = _TOTAL // _NW
_CH = 800
_NCH = _PER_W // _CH
_SUBS = [(o, min(128, _CH - o)) for o in range(0, _CH, 128)]


def _body(idx_hbm, wt_hbm, pos_hbm, out_hbm,
          idx_v, rows_v, pos_v, gsem, osem):
    wid = lax.axis_index("s") * 2 + lax.axis_index("c")
    base = wid * _PER_W
    pltpu.sync_copy(pos_hbm, pos_v)

    def chunk(g, carry):
        row0 = base + g * _CH
        pltpu.sync_copy(idx_hbm.at[pl.ds(row0, _CH)], idx_v)
        copies = [
            pltpu.async_copy(wt_hbm.at[idx_v.at[pl.ds(o, n)]],
                             rows_v.at[pl.ds(o, n)], gsem)
            for o, n in _SUBS
        ]
        for c in copies:
            c.wait()

        def addl(l, c2):
            p0 = pos_v[l, pl.ds(0, 16)]
            p1 = pos_v[l, pl.ds(16, 16)]
            for s in range(_CH // _SEQ):
                r = s * _SEQ + l
                rows_v[r, pl.ds(0, 16)] = rows_v[r, pl.ds(0, 16)] + p0
                rows_v[r, pl.ds(16, 16)] = rows_v[r, pl.ds(16, 16)] + p1
            return c2

        lax.fori_loop(0, _SEQ, addl, 0)
        pltpu.async_copy(rows_v, out_hbm.at[pl.ds(row0, _CH)], osem).wait()
        return carry

    lax.fori_loop(0, _NCH, chunk, 0)


_emb = functools.partial(
    pl.kernel,
    out_type=jax.ShapeDtypeStruct((_TOTAL, _DIM), jnp.float32),
    mesh=plsc.VectorSubcoreMesh(core_axis_name="c", subcore_axis_name="s"),
    compiler_params=pltpu.CompilerParams(use_tc_tiling_on_sc=False),
    scratch_types=[
        pltpu.VMEM((_CH,), jnp.int32),
        pltpu.VMEM((_CH, _DIM), jnp.float32),
        pltpu.VMEM((_SEQ, _DIM), jnp.float32),
        pltpu.SemaphoreType.DMA,
        pltpu.SemaphoreType.DMA,
    ],
)(_body)


def kernel(inputs, word_table, pos_table):
    flat = inputs.reshape(_TOTAL).astype(jnp.int32)
    out = _emb(flat, word_table, pos_table)
    return out.reshape(_BATCH, _SEQ, _DIM)
